# baseline (speedup 1.0000x reference)
import functools

import jax
import jax.numpy as jnp
from jax import lax
from jax.experimental import pallas as pl
from jax.experimental.pallas import tpu as pltpu
from jax.experimental.pallas import tpu_sc as plsc

NUM_ROWS = 1_000_000
DIM = 32
GRP = 128 // DIM
B_TOTAL = 4096 * 200

_info = plsc.get_sparse_core_info()
NC, NS, NL = _info.num_cores, _info.num_subcores, _info.num_lanes
NW = NC * NS
PER_W = B_TOTAL // NW
CHUNK = 320
N_CHUNKS = PER_W // CHUNK
D_BUF = 2


def _make_gather():
    mesh = plsc.VectorSubcoreMesh(core_axis_name="c", subcore_axis_name="s")

    @functools.partial(
        pl.kernel,
        mesh=mesh,
        out_type=jax.ShapeDtypeStruct((B_TOTAL // GRP, 128), jnp.float32),
        scratch_types=(
            [pltpu.VMEM((CHUNK,), jnp.int32)] * D_BUF
            + [pltpu.VMEM((CHUNK,), jnp.int32)] * D_BUF
            + [pltpu.VMEM((CHUNK, 128), jnp.float32)] * D_BUF
            + [pltpu.VMEM((CHUNK // GRP, 128), jnp.float32)] * D_BUF
            + [pltpu.SemaphoreType.DMA] * (3 * D_BUF)
        ),
        compiler_params=pltpu.CompilerParams(
            use_tc_tiling_on_sc=True, needs_layout_passes=False),
    )
    def gather(table_hbm, idx_hbm, out_hbm, *scr):
        idx_v = scr[0:D_BUF]
        idx4_v = scr[D_BUF:2 * D_BUF]
        rows_v = scr[2 * D_BUF:3 * D_BUF]
        out_v = scr[3 * D_BUF:4 * D_BUF]
        i_sem = scr[4 * D_BUF:5 * D_BUF]
        g_sem = scr[5 * D_BUF:6 * D_BUF]
        o_sem = scr[6 * D_BUF:7 * D_BUF]
        wid = lax.axis_index("s") * NC + lax.axis_index("c")
        w_base = wid * PER_W

        def idx_copy(k, b):
            off = pl.multiple_of(w_base + k * CHUNK, CHUNK)
            return pltpu.make_async_copy(
                idx_hbm.at[pl.ds(off, CHUNK)], idx_v[b], i_sem[b])

        def gather_copy(b):
            return pltpu.make_async_copy(
                table_hbm.at[idx4_v[b]], rows_v[b], g_sem[b])

        def out_copy(k, b):
            off = pl.multiple_of((w_base + k * CHUNK) // GRP, CHUNK // GRP)
            return pltpu.make_async_copy(
                out_v[b],
                out_hbm.at[pl.ds(off, CHUNK // GRP)], o_sem[b])

        def prep_idx(b):
            def t_body(t, _):
                v = idx_v[b][pl.ds(t * NL, NL)]
                idx4_v[b][pl.ds(t * NL, NL)] = lax.shift_right_logical(v, 2)
                return 0
            lax.fori_loop(0, CHUNK // NL, t_body, 0)

        def extract(b):
            def t_body(t, _):
                j_vec = t * NL + lax.iota(jnp.int32, NL)
                v = idx_v[b][pl.ds(t * NL, NL)]
                src_base = lax.rem(v, 4) * DIM
                dst_row = lax.shift_right_logical(j_vec, 2)
                dst_base = lax.rem(j_vec, 4) * DIM
                for c in range(DIM):
                    vals = plsc.load_gather(rows_v[b], [j_vec, src_base + c])
                    plsc.store_scatter(out_v[b], [dst_row, dst_base + c], vals)
                return 0
            lax.fori_loop(0, CHUNK // NL, t_body, 0)

        for b in range(D_BUF):
            idx_copy(b, b).start()
        idx_copy(0, 0).wait()
        prep_idx(0)
        gather_copy(0).start()

        def pair_body(g, _):
            for b in range(D_BUF):
                k = g * D_BUF + b
                p = (b + 1) % D_BUF

                @pl.when(k <= N_CHUNKS - 2)
                def _():
                    idx_copy(k + 1, p).wait()
                    prep_idx(p)
                    @pl.when(k >= 1)
                    def _():
                        out_copy(k - 1, p).wait()
                    gather_copy(p).start()

                gather_copy(b).wait()
                extract(b)
                out_copy(k, b).start()

                @pl.when(k <= N_CHUNKS - 3)
                def _():
                    idx_copy(k + 2, b).start()
            return 0

        lax.fori_loop(0, N_CHUNKS // D_BUF, pair_body, 0)

        out_copy(N_CHUNKS - 2, (N_CHUNKS - 2) % D_BUF).wait()
        out_copy(N_CHUNKS - 1, (N_CHUNKS - 1) % D_BUF).wait()

    return gather


_gather = _make_gather()


def kernel(x, weight):
    B, L = x.shape
    flat = x.reshape(-1).astype(jnp.int32)
    table4 = weight.reshape(NUM_ROWS // GRP, 128)
    out = _gather(table4, flat)
    return out.reshape(B, L, DIM)


# speedup vs baseline: 1.9437x; 1.9437x over previous
import functools

import jax
import jax.numpy as jnp
from jax import lax
from jax.experimental import pallas as pl
from jax.experimental.pallas import tpu as pltpu
from jax.experimental.pallas import tpu_sc as plsc

NUM_ROWS = 1_000_000
DIM = 32
B_TOTAL = 4096 * 200

_info = plsc.get_sparse_core_info()
NC, NS = _info.num_cores, _info.num_subcores
NW = NC * NS
PER_W = B_TOTAL // NW
CHUNK = 1600
N_CHUNKS = PER_W // CHUNK
D_BUF = 2


def _make_gather():
    mesh = plsc.VectorSubcoreMesh(core_axis_name="c", subcore_axis_name="s")

    @functools.partial(
        pl.kernel,
        mesh=mesh,
        out_type=jax.ShapeDtypeStruct((B_TOTAL, DIM), jnp.float32),
        scratch_types=[
            pltpu.VMEM((D_BUF, CHUNK), jnp.int32),
            pltpu.VMEM((D_BUF, CHUNK, DIM), jnp.float32),
            pltpu.SemaphoreType.DMA((D_BUF,)),
            pltpu.SemaphoreType.DMA((D_BUF,)),
            pltpu.SemaphoreType.DMA((D_BUF,)),
        ],
        compiler_params=pltpu.CompilerParams(use_tc_tiling_on_sc=False),
    )
    def gather(table_hbm, idx_hbm, out_hbm, idx_v, rows_v, i_sem, g_sem, o_sem):
        wid = lax.axis_index("s") * NC + lax.axis_index("c")
        w_base = wid * PER_W

        def idx_copy(k, b):
            return pltpu.make_async_copy(
                idx_hbm.at[pl.ds(w_base + k * CHUNK, CHUNK)],
                idx_v.at[b], i_sem.at[b])

        def gather_copy(b):
            return pltpu.make_async_copy(
                table_hbm.at[idx_v.at[b]], rows_v.at[b], g_sem.at[b])

        def out_copy(k, b):
            return pltpu.make_async_copy(
                rows_v.at[b],
                out_hbm.at[pl.ds(w_base + k * CHUNK, CHUNK)], o_sem.at[b])

        for b in range(D_BUF):
            idx_copy(b, b).start()

        def body(g, _):
            for b in range(D_BUF):
                k = g * D_BUF + b
                p = (b + 1) % D_BUF
                idx_copy(k, b).wait()

                @pl.when(k >= D_BUF)
                def _():
                    out_copy(k - D_BUF, b).wait()

                gather_copy(b).start()

                @pl.when(k >= 1)
                def _():
                    gather_copy(p).wait()
                    out_copy(k - 1, p).start()

                @pl.when((k >= 1) & (k <= N_CHUNKS - 2))
                def _():
                    idx_copy(k + 1, p).start()

            return 0

        lax.fori_loop(0, N_CHUNKS // D_BUF, body, 0)

        last = N_CHUNKS - 1
        bl = last % D_BUF
        gather_copy(bl).wait()
        out_copy(last, bl).start()
        out_copy(last - 1, (last - 1) % D_BUF).wait()
        out_copy(last, bl).wait()

    return gather


_gather = _make_gather()


def kernel(x, weight):
    B, L = x.shape
    flat = x.reshape(-1).astype(jnp.int32)
    out = _gather(weight, flat)
    return out.reshape(B, L, DIM)
